# baseline (speedup 1.0000x reference)
import functools

import jax
import jax.numpy as jnp
from jax import lax
from jax.experimental import pallas as pl
from jax.experimental.pallas import tpu as pltpu
from jax.experimental.pallas import tpu_sc as plsc

N = 10000
NPAD = 10240
N_GRAPHS = 64
DH = 64
NC = 2
NS = 16
NW = NC * NS
RPS = NPAD // NS
C = 128
E_TOT = 320000 + N
K = -(-E_TOT // (NW * C))
EP = NW * C * K
R = 1024
GRID = NPAD // R
NB = 3
K0 = 81
K1 = 2 * K - K0
MAXK = max(K0, K1)
NROW2 = NW * K + MAXK

_mesh = plsc.VectorSubcoreMesh(core_axis_name="c", subcore_axis_name="s")
_sc_params = pltpu.CompilerParams(use_tc_tiling_on_sc=False)


@functools.partial(
    pl.kernel,
    out_type=jax.ShapeDtypeStruct((NC * NPAD, 16), jnp.float32),
    mesh=_mesh,
    scratch_types=[
        pltpu.VMEM((C, 16), jnp.float32),
        pltpu.VMEM((K, C), jnp.int32),
        pltpu.VMEM_SHARED((NPAD, 16), jnp.float32),
    ],
    compiler_params=_sc_params,
)
def _sc_deg(dst2_hbm, out_hbm, buf_v, didx, acc_sh):
    c = lax.axis_index("c")
    s = lax.axis_index("s")
    wid = c * NS + s

    def _fill(val):
        def row(i, _):
            buf_v[i] = jnp.full((16,), val, jnp.float32)
            return 0
        lax.fori_loop(0, C, row, 0)

    _fill(0.0)
    for t in range(RPS // C):
        pltpu.sync_copy(buf_v, acc_sh.at[pl.ds(s * RPS + t * C, C)])
    pltpu.sync_copy(dst2_hbm.at[pl.ds(wid * K, K)], didx)
    plsc.subcore_barrier()
    _fill(1.0)

    def chunk(k, _):
        pltpu.sync_copy(buf_v, acc_sh.at[didx.at[k]], add=True)
        return 0
    lax.fori_loop(0, K, chunk, 0)
    plsc.subcore_barrier()
    pltpu.sync_copy(acc_sh.at[pl.ds(s * RPS, RPS)],
                    out_hbm.at[pl.ds(c * NPAD + s * RPS, RPS)])


@functools.partial(
    pl.kernel,
    out_type=jax.ShapeDtypeStruct((NPAD, 128), jnp.float32),
    mesh=_mesh,
    scratch_types=[
        pltpu.VMEM((MAXK, C), jnp.int32),
        pltpu.VMEM((MAXK, C), jnp.int32),
        pltpu.VMEM((NB, C, DH), jnp.float32),
        pltpu.VMEM((C, DH), jnp.float32),
        pltpu.VMEM_SHARED((NPAD, DH), jnp.float32),
    ] + [pltpu.SemaphoreType.DMA] * NB,
    compiler_params=_sc_params,
)
def _sc_spmm(hs_hbm, src2_hbm, dst2_hbm, out_hbm, sidx, didx, rows_v, zb,
             acc_sh, *sems):
    c = lax.axis_index("c")
    s = lax.axis_index("s")
    kc = jnp.where(c == 0, K0, K1)
    base_row = jnp.where(c == 0, s * K0, NS * K0 + s * K1)

    def zrow(i, _):
        for j in range(DH // 16):
            zb[i, pl.ds(j * 16, 16)] = jnp.zeros((16,), jnp.float32)
        return 0
    lax.fori_loop(0, C, zrow, 0)
    for t in range(RPS // C):
        pltpu.sync_copy(zb, acc_sh.at[pl.ds(s * RPS + t * C, C)])
    pltpu.sync_copy(src2_hbm.at[pl.ds(base_row, MAXK)], sidx)
    pltpu.sync_copy(dst2_hbm.at[pl.ds(base_row, MAXK)], didx)
    plsc.subcore_barrier()

    for b in range(NB):
        pltpu.async_copy(hs_hbm.at[sidx.at[b]], rows_v.at[b], sems[b])

    def group(i, _):
        for b in range(NB):
            k = i * NB + b
            pltpu.make_async_copy(hs_hbm.at[sidx.at[b]], rows_v.at[b],
                                  sems[b]).wait()
            pltpu.sync_copy(rows_v.at[b], acc_sh.at[didx.at[k]], add=True)
            pltpu.async_copy(hs_hbm.at[sidx.at[k + NB]], rows_v.at[b],
                             sems[b])
        return 0
    lax.fori_loop(0, kc // NB - 1, group, 0)
    for b in range(NB):
        k = kc - NB + b
        pltpu.make_async_copy(hs_hbm.at[sidx.at[b]], rows_v.at[b],
                              sems[b]).wait()
        pltpu.sync_copy(rows_v.at[b], acc_sh.at[didx.at[k]], add=True)
    plsc.subcore_barrier()
    pltpu.sync_copy(acc_sh.at[pl.ds(s * RPS, RPS)],
                    out_hbm.at[pl.ds(s * RPS, RPS), pl.ds(c * DH, DH)])


def _dinv_of(d0, d1):
    deg = d0[:, :1] + d1[:, :1]
    return jnp.where(deg > 0, lax.rsqrt(deg), 0.0)


def _tc0_body(x_ref, W_ref, d0_ref, d1_ref, hs_ref):
    dinv = _dinv_of(d0_ref[...], d1_ref[...])
    hs_ref[:, :DH] = jnp.dot(x_ref[...], W_ref[...],
                             preferred_element_type=jnp.float32) * dinv


def _tc0(x_p, W0, degf):
    return pl.pallas_call(
        _tc0_body,
        grid=(GRID,),
        in_specs=[
            pl.BlockSpec((R, 128), lambda i: (i, 0)),
            pl.BlockSpec((128, DH), lambda i: (0, 0)),
            pl.BlockSpec((R, 16), lambda i: (i, 0)),
            pl.BlockSpec((R, 16), lambda i: (i + GRID, 0)),
        ],
        out_specs=pl.BlockSpec((R, 128), lambda i: (i, 0)),
        out_shape=jax.ShapeDtypeStruct((NPAD, 128), jnp.float32),
    )(x_p, W0, degf, degf)


def _tc12_body(pf_ref, d0_ref, d1_ref, b_ref, W_ref, jk_ref, hs_ref):
    dinv = _dinv_of(d0_ref[...], d1_ref[...])
    acc = pf_ref[:, :DH] + pf_ref[:, DH:]
    jkv = jnp.maximum(acc * dinv + b_ref[...], 0.0)
    jk_ref[...] = jkv
    hs_ref[:, :DH] = jnp.dot(jkv, W_ref[...],
                             preferred_element_type=jnp.float32) * dinv


def _tc12(pf, degf, b, W):
    return pl.pallas_call(
        _tc12_body,
        grid=(GRID,),
        in_specs=[
            pl.BlockSpec((R, 128), lambda i: (i, 0)),
            pl.BlockSpec((R, 16), lambda i: (i, 0)),
            pl.BlockSpec((R, 16), lambda i: (i + GRID, 0)),
            pl.BlockSpec((1, DH), lambda i: (0, 0)),
            pl.BlockSpec((DH, DH), lambda i: (0, 0)),
        ],
        out_specs=[
            pl.BlockSpec((R, DH), lambda i: (i, 0)),
            pl.BlockSpec((R, 128), lambda i: (i, 0)),
        ],
        out_shape=[
            jax.ShapeDtypeStruct((NPAD, DH), jnp.float32),
            jax.ShapeDtypeStruct((NPAD, 128), jnp.float32),
        ],
    )(pf, degf, degf, b.reshape(1, DH), W)


def _ln(x, gamma, beta, eps=1e-5):
    mu = jnp.mean(x, axis=-1, keepdims=True)
    var = jnp.mean((x - mu) ** 2, axis=-1, keepdims=True)
    return (x - mu) / jnp.sqrt(var + eps) * gamma + beta


def _tc3_body(pf_ref, d0_ref, d1_ref, b_ref, jk0_ref,
              jk1_ref, ids_ref, pi_ref, Wg_ref, bg_ref, gg_ref, bg2_ref,
              Wp_ref, bp_ref, gp_ref, bp2_ref, Wf_ref, bf_ref, gf_ref,
              bf2_ref, pooled_ref, out_ref):
    i = pl.program_id(0)
    dinv = _dinv_of(d0_ref[...], d1_ref[...])
    jk2 = jnp.maximum((pf_ref[:, :DH] + pf_ref[:, DH:]) * dinv + b_ref[...],
                      0.0)
    blk = jnp.concatenate([jk0_ref[...], jk1_ref[...], jk2], axis=1)
    ids = ids_ref[...]
    ninf = jnp.float32(-jnp.inf)
    giota = lax.broadcasted_iota(jnp.int32, (N_GRAPHS, 3 * DH), 0)

    @pl.when(i == 0)
    def _():
        pooled_ref[...] = jnp.full((N_GRAPHS, 3 * DH), ninf, jnp.float32)

    def body(g, accv):
        v = jnp.max(jnp.where(ids == g, blk, ninf), axis=0, keepdims=True)
        return jnp.where(giota == g, jnp.maximum(accv, v), accv)

    lo = jnp.min(ids)
    hi = jnp.max(jnp.where(ids == N_GRAPHS, -1, ids))
    acc = lax.fori_loop(
        lo, hi + 1, body,
        jnp.full((N_GRAPHS, 3 * DH), ninf, jnp.float32))
    pooled_ref[...] = jnp.maximum(pooled_ref[...], acc)

    @pl.when(i == GRID - 1)
    def _():
        g = jnp.dot(pooled_ref[...], Wg_ref[...],
                    preferred_element_type=jnp.float32)
        g = jax.nn.relu(_ln(g + bg_ref[...], gg_ref[...], bg2_ref[...]))
        p = jnp.dot(pi_ref[...], Wp_ref[...],
                    preferred_element_type=jnp.float32)
        p = jax.nn.relu(_ln(p + bp_ref[...], gp_ref[...], bp2_ref[...]))
        cat = jnp.concatenate([g, p], axis=1)
        o = jnp.dot(cat, Wf_ref[...], preferred_element_type=jnp.float32)
        out_ref[...] = _ln(o + bf_ref[...], gf_ref[...], bf2_ref[...])


def _tc3(pf, degf, b, jk0, jk1, ids, pi, Wg, bg, gg, bg2, Wp, bp, gp,
         bp2, Wf, bf, gf, bf2):
    r1 = lambda a: a.reshape(1, -1)
    full = lambda shp: pl.BlockSpec(shp, lambda i: (0, 0))
    _, out = pl.pallas_call(
        _tc3_body,
        grid=(GRID,),
        in_specs=[
            pl.BlockSpec((R, 128), lambda i: (i, 0)),
            pl.BlockSpec((R, 16), lambda i: (i, 0)),
            pl.BlockSpec((R, 16), lambda i: (i + GRID, 0)),
            full((1, DH)),
            pl.BlockSpec((R, DH), lambda i: (i, 0)),
            pl.BlockSpec((R, DH), lambda i: (i, 0)),
            pl.BlockSpec((R, 1), lambda i: (i, 0)),
            full(pi.shape), full(Wg.shape), full((1, bg.shape[0])),
            full((1, gg.shape[0])), full((1, bg2.shape[0])),
            full(Wp.shape), full((1, bp.shape[0])),
            full((1, gp.shape[0])), full((1, bp2.shape[0])),
            full(Wf.shape), full((1, bf.shape[0])),
            full((1, gf.shape[0])), full((1, bf2.shape[0])),
        ],
        out_specs=[
            pl.BlockSpec((N_GRAPHS, 3 * DH), lambda i: (0, 0)),
            pl.BlockSpec((N_GRAPHS, Wf.shape[1]), lambda i: (0, 0)),
        ],
        out_shape=[
            jax.ShapeDtypeStruct((N_GRAPHS, 3 * DH), jnp.float32),
            jax.ShapeDtypeStruct((N_GRAPHS, Wf.shape[1]), jnp.float32),
        ],
    )(pf, degf, degf, b.reshape(1, DH), jk0, jk1, ids, pi, Wg,
      r1(bg), r1(gg), r1(bg2), Wp, r1(bp), r1(gp), r1(bp2), Wf, r1(bf),
      r1(gf), r1(bf2))
    return out


def kernel(x, edge_index, batch, pi, Wc0, bc0, Wc1, bc1, Wc2, bc2, Wg, bg,
           gg, bg2, Wp, bp, gp, bp2, Wf, bf, gf, bf2):
    n = x.shape[0]
    loop = jnp.arange(n, dtype=edge_index.dtype)
    padi = jnp.full((NROW2 * C - E_TOT,), NPAD - 1, jnp.int32)
    prep = lambda v: jnp.concatenate([v, padi]).reshape(NROW2, C)
    src2 = prep(jnp.concatenate([edge_index[0], loop])) * 2
    dst2 = prep(jnp.concatenate([edge_index[1], loop]))
    x_p = jnp.pad(x, ((0, NPAD - n), (0, 0)))
    ids = jnp.concatenate(
        [batch, jnp.full((NPAD - n,), N_GRAPHS, jnp.int32)]).reshape(NPAD, 1)

    lin = lambda a: a.reshape(2 * NPAD, DH)
    degf = _sc_deg(dst2)
    hs = _tc0(x_p, Wc0, degf)
    pf = _sc_spmm(lin(hs), src2, dst2)
    jk0, hs = _tc12(pf, degf, bc0, Wc1)
    pf = _sc_spmm(lin(hs), src2, dst2)
    jk1, hs = _tc12(pf, degf, bc1, Wc2)
    pf = _sc_spmm(lin(hs), src2, dst2)
    return _tc3(pf, degf, bc2, jk0, jk1, ids, pi, Wg, bg, gg, bg2, Wp,
                bp, gp, bp2, Wf, bf, gf, bf2)


# speedup vs baseline: 1.0062x; 1.0062x over previous
import functools

import jax
import jax.numpy as jnp
from jax import lax
from jax.experimental import pallas as pl
from jax.experimental.pallas import tpu as pltpu
from jax.experimental.pallas import tpu_sc as plsc

N = 10000
NPAD = 10240
N_GRAPHS = 64
DH = 64
NC = 2
NS = 16
NW = NC * NS
RPS = NPAD // NS
C = 128
E_TOT = 320000 + N
K = -(-E_TOT // (NW * C))
EP = NW * C * K
R = 1024
GRID = NPAD // R
NB = 3
K0 = 81
K1 = 2 * K - K0
MAXK = max(K0, K1)
NROW2 = NW * K + MAXK

_mesh = plsc.VectorSubcoreMesh(core_axis_name="c", subcore_axis_name="s")
_sc_params = pltpu.CompilerParams(use_tc_tiling_on_sc=False)


@functools.partial(
    pl.kernel,
    out_type=jax.ShapeDtypeStruct((NC * NPAD, 16), jnp.float32),
    mesh=_mesh,
    scratch_types=[
        pltpu.VMEM((C, 16), jnp.float32),
        pltpu.VMEM((K, C), jnp.int32),
        pltpu.VMEM_SHARED((NPAD, 16), jnp.float32),
    ],
    compiler_params=_sc_params,
)
def _sc_deg(dst2_hbm, out_hbm, buf_v, didx, acc_sh):
    c = lax.axis_index("c")
    s = lax.axis_index("s")
    wid = c * NS + s

    def _fill(val):
        def row(i, _):
            buf_v[i] = jnp.full((16,), val, jnp.float32)
            return 0
        lax.fori_loop(0, C, row, 0)

    _fill(0.0)
    for t in range(RPS // C):
        pltpu.sync_copy(buf_v, acc_sh.at[pl.ds(s * RPS + t * C, C)])
    pltpu.sync_copy(dst2_hbm.at[pl.ds(wid * K, K)], didx)
    plsc.subcore_barrier()
    _fill(1.0)

    def chunk(k, _):
        pltpu.sync_copy(buf_v, acc_sh.at[didx.at[k]], add=True)
        return 0
    lax.fori_loop(0, K, chunk, 0)
    plsc.subcore_barrier()
    pltpu.sync_copy(acc_sh.at[pl.ds(s * RPS, RPS)],
                    out_hbm.at[pl.ds(c * NPAD + s * RPS, RPS)])


@functools.partial(
    pl.kernel,
    out_type=jax.ShapeDtypeStruct((NPAD, 128), jnp.float32),
    mesh=_mesh,
    scratch_types=[
        pltpu.VMEM((MAXK, C), jnp.int32),
        pltpu.VMEM((MAXK, C), jnp.int32),
        pltpu.VMEM((NB, C, DH), jnp.float32),
        pltpu.VMEM((C, DH), jnp.float32),
        pltpu.VMEM_SHARED((NPAD, DH), jnp.float32),
    ] + [pltpu.SemaphoreType.DMA] * NB,
    compiler_params=_sc_params,
)
def _sc_spmm(hs_hbm, src2_hbm, dst2_hbm, out_hbm, sidx, didx, rows_v, zb,
             acc_sh, *sems):
    c = lax.axis_index("c")
    s = lax.axis_index("s")
    kc = jnp.where(c == 0, K0, K1)
    base_row = jnp.where(c == 0, s * K0, NS * K0 + s * K1)

    def zrow(i, _):
        for j in range(DH // 16):
            zb[i, pl.ds(j * 16, 16)] = jnp.zeros((16,), jnp.float32)
        return 0
    lax.fori_loop(0, C, zrow, 0)
    for t in range(RPS // C):
        pltpu.sync_copy(zb, acc_sh.at[pl.ds(s * RPS + t * C, C)])
    pltpu.sync_copy(src2_hbm.at[pl.ds(base_row, MAXK)], sidx)
    pltpu.sync_copy(dst2_hbm.at[pl.ds(base_row, MAXK)], didx)
    plsc.subcore_barrier()

    for b in range(NB):
        pltpu.async_copy(hs_hbm.at[sidx.at[b]], rows_v.at[b], sems[b])

    def group(i, _):
        for b in range(NB):
            k = i * NB + b
            pltpu.make_async_copy(hs_hbm.at[sidx.at[b]], rows_v.at[b],
                                  sems[b]).wait()
            pltpu.sync_copy(rows_v.at[b], acc_sh.at[didx.at[k]], add=True)
            pltpu.async_copy(hs_hbm.at[sidx.at[k + NB]], rows_v.at[b],
                             sems[b])
        return 0
    lax.fori_loop(0, kc // NB - 1, group, 0)
    for b in range(NB):
        k = kc - NB + b
        pltpu.make_async_copy(hs_hbm.at[sidx.at[b]], rows_v.at[b],
                              sems[b]).wait()
        pltpu.sync_copy(rows_v.at[b], acc_sh.at[didx.at[k]], add=True)
    plsc.subcore_barrier()
    pltpu.sync_copy(acc_sh.at[pl.ds(s * RPS, RPS)],
                    out_hbm.at[pl.ds(s * RPS, RPS), pl.ds(c * DH, DH)])


def _dinv_of(d0, d1):
    deg = d0[:, :1] + d1[:, :1]
    return jnp.where(deg > 0, lax.rsqrt(deg), 0.0)


def _tc0_body(x_ref, W_ref, d0_ref, d1_ref, hs_ref):
    dinv = _dinv_of(d0_ref[...], d1_ref[...])
    hs_ref[:, :DH] = jnp.dot(x_ref[...], W_ref[...],
                             preferred_element_type=jnp.float32) * dinv


def _tc0(x_p, W0, degf):
    return pl.pallas_call(
        _tc0_body,
        grid=(GRID,),
        in_specs=[
            pl.BlockSpec((R, 128), lambda i: (i, 0)),
            pl.BlockSpec((128, DH), lambda i: (0, 0)),
            pl.BlockSpec((R, 16), lambda i: (i, 0)),
            pl.BlockSpec((R, 16), lambda i: (i + GRID, 0)),
        ],
        out_specs=pl.BlockSpec((R, 128), lambda i: (i, 0)),
        out_shape=jax.ShapeDtypeStruct((NPAD, 128), jnp.float32),
    )(x_p, W0, degf, degf)


def _tc12_body(pf_ref, d0_ref, d1_ref, b_ref, W_ref, jk_ref, hs_ref):
    dinv = _dinv_of(d0_ref[...], d1_ref[...])
    acc = pf_ref[:, :DH] + pf_ref[:, DH:]
    jkv = jnp.maximum(acc * dinv + b_ref[...], 0.0)
    jk_ref[...] = jkv
    hs_ref[:, :DH] = jnp.dot(jkv, W_ref[...],
                             preferred_element_type=jnp.float32) * dinv


def _tc12(pf, degf, b, W):
    return pl.pallas_call(
        _tc12_body,
        grid=(GRID,),
        in_specs=[
            pl.BlockSpec((R, 128), lambda i: (i, 0)),
            pl.BlockSpec((R, 16), lambda i: (i, 0)),
            pl.BlockSpec((R, 16), lambda i: (i + GRID, 0)),
            pl.BlockSpec((1, DH), lambda i: (0, 0)),
            pl.BlockSpec((DH, DH), lambda i: (0, 0)),
        ],
        out_specs=[
            pl.BlockSpec((R, DH), lambda i: (i, 0)),
            pl.BlockSpec((R, 128), lambda i: (i, 0)),
        ],
        out_shape=[
            jax.ShapeDtypeStruct((NPAD, DH), jnp.float32),
            jax.ShapeDtypeStruct((NPAD, 128), jnp.float32),
        ],
    )(pf, degf, degf, b.reshape(1, DH), W)


def _ln(x, gamma, beta, eps=1e-5):
    mu = jnp.mean(x, axis=-1, keepdims=True)
    var = jnp.mean((x - mu) ** 2, axis=-1, keepdims=True)
    return (x - mu) / jnp.sqrt(var + eps) * gamma + beta


def _tc3_body(pf_ref, d0_ref, d1_ref, b_ref, jk0_ref,
              jk1_ref, ids_ref, pi_ref, Wg_ref, bg_ref, gg_ref, bg2_ref,
              Wp_ref, bp_ref, gp_ref, bp2_ref, Wf_ref, bf_ref, gf_ref,
              bf2_ref, pooled_ref, out_ref):
    i = pl.program_id(0)
    dinv = _dinv_of(d0_ref[...], d1_ref[...])
    jk2 = jnp.maximum((pf_ref[:, :DH] + pf_ref[:, DH:]) * dinv + b_ref[...],
                      0.0)
    blk = jnp.concatenate([jk0_ref[...], jk1_ref[...], jk2], axis=1)
    ids = ids_ref[...]
    ninf = jnp.float32(-jnp.inf)
    giota = lax.broadcasted_iota(jnp.int32, (N_GRAPHS, 3 * DH), 0)

    @pl.when(i == 0)
    def _():
        pooled_ref[...] = jnp.full((N_GRAPHS, 3 * DH), ninf, jnp.float32)

    def body(g, accv):
        v = jnp.max(jnp.where(ids == g, blk, ninf), axis=0, keepdims=True)
        return jnp.where(giota == g, jnp.maximum(accv, v), accv)

    lo = ids_ref[0, 0]
    hi = ids_ref[R - 1, 0]
    acc = lax.fori_loop(
        lo, hi + 1, body,
        jnp.full((N_GRAPHS, 3 * DH), ninf, jnp.float32))
    pooled_ref[...] = jnp.maximum(pooled_ref[...], acc)

    @pl.when(i == GRID - 1)
    def _():
        g = jnp.dot(pooled_ref[...], Wg_ref[...],
                    preferred_element_type=jnp.float32)
        g = jax.nn.relu(_ln(g + bg_ref[...], gg_ref[...], bg2_ref[...]))
        p = jnp.dot(pi_ref[...], Wp_ref[...],
                    preferred_element_type=jnp.float32)
        p = jax.nn.relu(_ln(p + bp_ref[...], gp_ref[...], bp2_ref[...]))
        cat = jnp.concatenate([g, p], axis=1)
        o = jnp.dot(cat, Wf_ref[...], preferred_element_type=jnp.float32)
        out_ref[...] = _ln(o + bf_ref[...], gf_ref[...], bf2_ref[...])


def _tc3(pf, degf, b, jk0, jk1, ids, pi, Wg, bg, gg, bg2, Wp, bp, gp,
         bp2, Wf, bf, gf, bf2):
    r1 = lambda a: a.reshape(1, -1)
    full = lambda shp: pl.BlockSpec(shp, lambda i: (0, 0))
    _, out = pl.pallas_call(
        _tc3_body,
        grid=(GRID,),
        in_specs=[
            pl.BlockSpec((R, 128), lambda i: (i, 0)),
            pl.BlockSpec((R, 16), lambda i: (i, 0)),
            pl.BlockSpec((R, 16), lambda i: (i + GRID, 0)),
            full((1, DH)),
            pl.BlockSpec((R, DH), lambda i: (i, 0)),
            pl.BlockSpec((R, DH), lambda i: (i, 0)),
            pl.BlockSpec((R, 1), lambda i: (i, 0)),
            full(pi.shape), full(Wg.shape), full((1, bg.shape[0])),
            full((1, gg.shape[0])), full((1, bg2.shape[0])),
            full(Wp.shape), full((1, bp.shape[0])),
            full((1, gp.shape[0])), full((1, bp2.shape[0])),
            full(Wf.shape), full((1, bf.shape[0])),
            full((1, gf.shape[0])), full((1, bf2.shape[0])),
        ],
        out_specs=[
            pl.BlockSpec((N_GRAPHS, 3 * DH), lambda i: (0, 0)),
            pl.BlockSpec((N_GRAPHS, Wf.shape[1]), lambda i: (0, 0)),
        ],
        out_shape=[
            jax.ShapeDtypeStruct((N_GRAPHS, 3 * DH), jnp.float32),
            jax.ShapeDtypeStruct((N_GRAPHS, Wf.shape[1]), jnp.float32),
        ],
    )(pf, degf, degf, b.reshape(1, DH), jk0, jk1, ids, pi, Wg,
      r1(bg), r1(gg), r1(bg2), Wp, r1(bp), r1(gp), r1(bp2), Wf, r1(bf),
      r1(gf), r1(bf2))
    return out


def kernel(x, edge_index, batch, pi, Wc0, bc0, Wc1, bc1, Wc2, bc2, Wg, bg,
           gg, bg2, Wp, bp, gp, bp2, Wf, bf, gf, bf2):
    n = x.shape[0]
    loop = jnp.arange(n, dtype=edge_index.dtype)
    padi = jnp.full((NROW2 * C - E_TOT,), NPAD - 1, jnp.int32)
    prep = lambda v: jnp.concatenate([v, padi]).reshape(NROW2, C)
    src2 = prep(jnp.concatenate([edge_index[0], loop])) * 2
    dst2 = prep(jnp.concatenate([edge_index[1], loop]))
    x_p = jnp.pad(x, ((0, NPAD - n), (0, 0)))
    ids = jnp.concatenate(
        [batch, jnp.full((NPAD - n,), N_GRAPHS, jnp.int32)]).reshape(NPAD, 1)

    lin = lambda a: a.reshape(2 * NPAD, DH)
    degf = _sc_deg(dst2)
    hs = _tc0(x_p, Wc0, degf)
    pf = _sc_spmm(lin(hs), src2, dst2)
    jk0, hs = _tc12(pf, degf, bc0, Wc1)
    pf = _sc_spmm(lin(hs), src2, dst2)
    jk1, hs = _tc12(pf, degf, bc1, Wc2)
    pf = _sc_spmm(lin(hs), src2, dst2)
    return _tc3(pf, degf, bc2, jk0, jk1, ids, pi, Wg, bg, gg, bg2, Wp,
                bp, gp, bp2, Wf, bf, gf, bf2)


# speedup vs baseline: 1.0097x; 1.0035x over previous
import functools

import jax
import jax.numpy as jnp
from jax import lax
from jax.experimental import pallas as pl
from jax.experimental.pallas import tpu as pltpu
from jax.experimental.pallas import tpu_sc as plsc

N = 10000
NPAD = 10240
N_GRAPHS = 64
DH = 64
NC = 2
NS = 16
NW = NC * NS
RPS = NPAD // NS
C = 128
E_TOT = 320000 + N
K = -(-E_TOT // (NW * C))
EP = NW * C * K
R = 1024
GRID = NPAD // R
NB = 3
K0 = 81
K1 = 2 * K - K0
MAXK = max(K0, K1)
NROW2 = NW * K + MAXK

_mesh = plsc.VectorSubcoreMesh(core_axis_name="c", subcore_axis_name="s")
_sc_params = pltpu.CompilerParams(use_tc_tiling_on_sc=False)


@functools.partial(
    pl.kernel,
    out_type=jax.ShapeDtypeStruct((NC * NPAD, 16), jnp.float32),
    mesh=_mesh,
    scratch_types=[
        pltpu.VMEM((C, 16), jnp.float32),
        pltpu.VMEM((K, C), jnp.int32),
        pltpu.VMEM_SHARED((NPAD, 16), jnp.float32),
    ],
    compiler_params=_sc_params,
)
def _sc_deg(dst2_hbm, out_hbm, buf_v, didx, acc_sh):
    c = lax.axis_index("c")
    s = lax.axis_index("s")
    wid = c * NS + s

    def _fill(val):
        def row(i, _):
            buf_v[i] = jnp.full((16,), val, jnp.float32)
            return 0
        lax.fori_loop(0, C, row, 0)

    _fill(0.0)
    for t in range(RPS // C):
        pltpu.sync_copy(buf_v, acc_sh.at[pl.ds(s * RPS + t * C, C)])
    pltpu.sync_copy(dst2_hbm.at[pl.ds(wid * K, K)], didx)
    plsc.subcore_barrier()
    _fill(1.0)

    def chunk(k, _):
        pltpu.sync_copy(buf_v, acc_sh.at[didx.at[k]], add=True)
        return 0
    lax.fori_loop(0, K, chunk, 0)
    plsc.subcore_barrier()
    pltpu.sync_copy(acc_sh.at[pl.ds(s * RPS, RPS)],
                    out_hbm.at[pl.ds(c * NPAD + s * RPS, RPS)])


@functools.partial(
    pl.kernel,
    out_type=jax.ShapeDtypeStruct((NC * NPAD, 128), jnp.float32),
    mesh=_mesh,
    scratch_types=[
        pltpu.VMEM((MAXK, C), jnp.int32),
        pltpu.VMEM((MAXK, C), jnp.int32),
        pltpu.VMEM((NB, C, DH), jnp.float32),
        pltpu.VMEM((C, DH), jnp.float32),
        pltpu.VMEM_SHARED((NPAD, DH), jnp.float32),
    ] + [pltpu.SemaphoreType.DMA] * NB,
    compiler_params=_sc_params,
)
def _sc_spmm(hs_hbm, src2_hbm, dst2_hbm, out_hbm, sidx, didx, rows_v, zb,
             acc_sh, *sems):
    c = lax.axis_index("c")
    s = lax.axis_index("s")
    kc = jnp.where(c == 0, K0, K1)
    base_row = jnp.where(c == 0, s * K0, NS * K0 + s * K1)

    def zrow(i, _):
        for j in range(DH // 16):
            zb[i, pl.ds(j * 16, 16)] = jnp.zeros((16,), jnp.float32)
        return 0
    lax.fori_loop(0, C, zrow, 0)
    for t in range(RPS // C):
        pltpu.sync_copy(zb, acc_sh.at[pl.ds(s * RPS + t * C, C)])
    pltpu.sync_copy(src2_hbm.at[pl.ds(base_row, MAXK)], sidx)
    pltpu.sync_copy(dst2_hbm.at[pl.ds(base_row, MAXK)], didx)
    plsc.subcore_barrier()

    for b in range(NB):
        pltpu.async_copy(hs_hbm.at[sidx.at[b]], rows_v.at[b], sems[b])

    def group(i, _):
        for b in range(NB):
            k = i * NB + b
            pltpu.make_async_copy(hs_hbm.at[sidx.at[b]], rows_v.at[b],
                                  sems[b]).wait()
            pltpu.sync_copy(rows_v.at[b], acc_sh.at[didx.at[k]], add=True)
            pltpu.async_copy(hs_hbm.at[sidx.at[k + NB]], rows_v.at[b],
                             sems[b])
        return 0
    lax.fori_loop(0, kc // NB - 1, group, 0)
    for b in range(NB):
        k = kc - NB + b
        pltpu.make_async_copy(hs_hbm.at[sidx.at[b]], rows_v.at[b],
                              sems[b]).wait()
        pltpu.sync_copy(rows_v.at[b], acc_sh.at[didx.at[k]], add=True)
    plsc.subcore_barrier()
    pltpu.sync_copy(acc_sh.at[pl.ds(s * RPS, RPS)],
                    out_hbm.at[pl.ds(c * NPAD + s * RPS, RPS), pl.ds(0, DH)])


def _dinv_of(d0, d1):
    deg = d0[:, :1] + d1[:, :1]
    return jnp.where(deg > 0, lax.rsqrt(deg), 0.0)


def _tc0_body(x_ref, W_ref, d0_ref, d1_ref, hs_ref):
    dinv = _dinv_of(d0_ref[...], d1_ref[...])
    hs_ref[:, :DH] = jnp.dot(x_ref[...], W_ref[...],
                             preferred_element_type=jnp.float32) * dinv


def _tc0(x_p, W0, degf):
    return pl.pallas_call(
        _tc0_body,
        grid=(GRID,),
        in_specs=[
            pl.BlockSpec((R, 128), lambda i: (i, 0)),
            pl.BlockSpec((128, DH), lambda i: (0, 0)),
            pl.BlockSpec((R, 16), lambda i: (i, 0)),
            pl.BlockSpec((R, 16), lambda i: (i + GRID, 0)),
        ],
        out_specs=pl.BlockSpec((R, 128), lambda i: (i, 0)),
        out_shape=jax.ShapeDtypeStruct((NPAD, 128), jnp.float32),
    )(x_p, W0, degf, degf)


def _tc12_body(p0_ref, p1_ref, d0_ref, d1_ref, b_ref, W_ref, jk_ref, hs_ref):
    dinv = _dinv_of(d0_ref[...], d1_ref[...])
    acc = p0_ref[:, :DH] + p1_ref[:, :DH]
    jkv = jnp.maximum(acc * dinv + b_ref[...], 0.0)
    jk_ref[...] = jkv
    hs_ref[:, :DH] = jnp.dot(jkv, W_ref[...],
                             preferred_element_type=jnp.float32) * dinv


def _tc12(pf, degf, b, W):
    return pl.pallas_call(
        _tc12_body,
        grid=(GRID,),
        in_specs=[
            pl.BlockSpec((R, 128), lambda i: (i, 0)),
            pl.BlockSpec((R, 128), lambda i: (i + GRID, 0)),
            pl.BlockSpec((R, 16), lambda i: (i, 0)),
            pl.BlockSpec((R, 16), lambda i: (i + GRID, 0)),
            pl.BlockSpec((1, DH), lambda i: (0, 0)),
            pl.BlockSpec((DH, DH), lambda i: (0, 0)),
        ],
        out_specs=[
            pl.BlockSpec((R, DH), lambda i: (i, 0)),
            pl.BlockSpec((R, 128), lambda i: (i, 0)),
        ],
        out_shape=[
            jax.ShapeDtypeStruct((NPAD, DH), jnp.float32),
            jax.ShapeDtypeStruct((NPAD, 128), jnp.float32),
        ],
    )(pf, pf, degf, degf, b.reshape(1, DH), W)


def _ln(x, gamma, beta, eps=1e-5):
    mu = jnp.mean(x, axis=-1, keepdims=True)
    var = jnp.mean((x - mu) ** 2, axis=-1, keepdims=True)
    return (x - mu) / jnp.sqrt(var + eps) * gamma + beta


def _tc3_body(lohi_ref, p0_ref, p1_ref, d0_ref, d1_ref, b_ref, jk0_ref,
              jk1_ref, ids_ref, pi_ref, Wg_ref, bg_ref, gg_ref, bg2_ref,
              Wp_ref, bp_ref, gp_ref, bp2_ref, Wf_ref, bf_ref, gf_ref,
              bf2_ref, pooled_ref, out_ref):
    i = pl.program_id(0)
    dinv = _dinv_of(d0_ref[...], d1_ref[...])
    jk2 = jnp.maximum((p0_ref[:, :DH] + p1_ref[:, :DH]) * dinv + b_ref[...],
                      0.0)
    blk = jnp.concatenate([jk0_ref[...], jk1_ref[...], jk2], axis=1)
    ids = ids_ref[...]
    ninf = jnp.float32(-jnp.inf)
    giota = lax.broadcasted_iota(jnp.int32, (N_GRAPHS, 3 * DH), 0)

    @pl.when(i == 0)
    def _():
        pooled_ref[...] = jnp.full((N_GRAPHS, 3 * DH), ninf, jnp.float32)

    def body(g, accv):
        v = jnp.max(jnp.where(ids == g, blk, ninf), axis=0, keepdims=True)
        return jnp.where(giota == g, jnp.maximum(accv, v), accv)

    acc = lax.fori_loop(
        lohi_ref[0, i], lohi_ref[1, i] + 1, body,
        jnp.full((N_GRAPHS, 3 * DH), ninf, jnp.float32))
    pooled_ref[...] = jnp.maximum(pooled_ref[...], acc)

    @pl.when(i == GRID - 1)
    def _():
        g = jnp.dot(pooled_ref[...], Wg_ref[...],
                    preferred_element_type=jnp.float32)
        g = jax.nn.relu(_ln(g + bg_ref[...], gg_ref[...], bg2_ref[...]))
        p = jnp.dot(pi_ref[...], Wp_ref[...],
                    preferred_element_type=jnp.float32)
        p = jax.nn.relu(_ln(p + bp_ref[...], gp_ref[...], bp2_ref[...]))
        cat = jnp.concatenate([g, p], axis=1)
        o = jnp.dot(cat, Wf_ref[...], preferred_element_type=jnp.float32)
        out_ref[...] = _ln(o + bf_ref[...], gf_ref[...], bf2_ref[...])


def _tc3(lohi, pf, degf, b, jk0, jk1, ids, pi, Wg, bg, gg, bg2, Wp, bp, gp,
         bp2, Wf, bf, gf, bf2):
    r1 = lambda a: a.reshape(1, -1)
    full = lambda shp: pl.BlockSpec(shp, lambda i, s: (0, 0))
    _, out = pl.pallas_call(
        _tc3_body,
        grid_spec=pltpu.PrefetchScalarGridSpec(
            num_scalar_prefetch=1,
            grid=(GRID,),
            in_specs=[
                pl.BlockSpec((R, 128), lambda i, s: (i, 0)),
                pl.BlockSpec((R, 128), lambda i, s: (i + GRID, 0)),
                pl.BlockSpec((R, 16), lambda i, s: (i, 0)),
                pl.BlockSpec((R, 16), lambda i, s: (i + GRID, 0)),
                full((1, DH)),
                pl.BlockSpec((R, DH), lambda i, s: (i, 0)),
                pl.BlockSpec((R, DH), lambda i, s: (i, 0)),
                pl.BlockSpec((R, 1), lambda i, s: (i, 0)),
                full(pi.shape), full(Wg.shape), full((1, bg.shape[0])),
                full((1, gg.shape[0])), full((1, bg2.shape[0])),
                full(Wp.shape), full((1, bp.shape[0])),
                full((1, gp.shape[0])), full((1, bp2.shape[0])),
                full(Wf.shape), full((1, bf.shape[0])),
                full((1, gf.shape[0])), full((1, bf2.shape[0])),
            ],
            out_specs=[
                pl.BlockSpec((N_GRAPHS, 3 * DH), lambda i, s: (0, 0)),
                pl.BlockSpec((N_GRAPHS, Wf.shape[1]), lambda i, s: (0, 0)),
            ],
        ),
        out_shape=[
            jax.ShapeDtypeStruct((N_GRAPHS, 3 * DH), jnp.float32),
            jax.ShapeDtypeStruct((N_GRAPHS, Wf.shape[1]), jnp.float32),
        ],
    )(lohi, pf, pf, degf, degf, b.reshape(1, DH), jk0, jk1, ids, pi, Wg,
      r1(bg), r1(gg), r1(bg2), Wp, r1(bp), r1(gp), r1(bp2), Wf, r1(bf),
      r1(gf), r1(bf2))
    return out


def kernel(x, edge_index, batch, pi, Wc0, bc0, Wc1, bc1, Wc2, bc2, Wg, bg,
           gg, bg2, Wp, bp, gp, bp2, Wf, bf, gf, bf2):
    n = x.shape[0]
    loop = jnp.arange(n, dtype=edge_index.dtype)
    padi = jnp.full((NROW2 * C - E_TOT,), NPAD - 1, jnp.int32)
    prep = lambda v: jnp.concatenate([v, padi]).reshape(NROW2, C)
    src2 = prep(jnp.concatenate([edge_index[0], loop])) * 2
    dst2 = prep(jnp.concatenate([edge_index[1], loop]))
    x_p = jnp.pad(x, ((0, NPAD - n), (0, 0)))
    ids_f = jnp.concatenate(
        [batch, jnp.full((NPAD - n,), N_GRAPHS, jnp.int32)])
    ids = ids_f.reshape(NPAD, 1)
    lo = ids_f[::R]
    hi = ids_f[jnp.minimum(jnp.arange(GRID) * R + R - 1, n - 1)]
    lohi = jnp.stack([lo, hi]).astype(jnp.int32)

    lin = lambda a: a.reshape(2 * NPAD, DH)
    degf = _sc_deg(dst2)
    hs = _tc0(x_p, Wc0, degf)
    pf = _sc_spmm(lin(hs), src2, dst2)
    jk0, hs = _tc12(pf, degf, bc0, Wc1)
    pf = _sc_spmm(lin(hs), src2, dst2)
    jk1, hs = _tc12(pf, degf, bc1, Wc2)
    pf = _sc_spmm(lin(hs), src2, dst2)
    return _tc3(lohi, pf, degf, bc2, jk0, jk1, ids, pi, Wg, bg, gg, bg2, Wp,
                bp, gp, bp2, Wf, bf, gf, bf2)
